# Initial kernel scaffold; baseline (speedup 1.0000x reference)
import functools

import jax
import jax.numpy as jnp
from jax import lax
from jax.experimental import pallas as pl
from jax.experimental.pallas import tpu as pltpu
from jax.experimental.pallas import tpu_sc as plsc

N_NODES = 10000
N_EDGES = 320000
F = 144
NC, NS = 2, 16
NW = NC * NS
E_PER_W = N_EDGES // NW
CH = 80
NCH = E_PER_W // CH
ROWS_PER_TILE = N_NODES // NS
RCH = 125
NRCH = ROWS_PER_TILE // RCH

_mesh = plsc.VectorSubcoreMesh(core_axis_name="c", subcore_axis_name="s")


def _zero_rows(buf, nrows, ncolv):
    def body(i, carry):
        for c in range(ncolv):
            buf[i, pl.ds(c * 16, 16)] = jnp.zeros((16,), jnp.float32)
        return carry
    lax.fori_loop(0, nrows, body, 0)


@functools.partial(
    pl.kernel,
    mesh=_mesh,
    out_type=jax.ShapeDtypeStruct((NC, N_NODES, 16), jnp.float32),
    scratch_types=[
        pltpu.VMEM_SHARED((N_NODES, 16), jnp.float32),
        pltpu.VMEM((CH, 16), jnp.float32),
        pltpu.VMEM((RCH, 16), jnp.float32),
        pltpu.VMEM((CH,), jnp.int32),
    ],
)
def _sc_degree(dst_hbm, out_hbm, acc_sh, ones_v, zero_v, idx_v):
    c = lax.axis_index("c")
    s = lax.axis_index("s")
    def ones_body(i, carry):
        ones_v[i, pl.ds(0, 16)] = jnp.full((16,), 1.0, jnp.float32)
        return carry
    lax.fori_loop(0, CH, ones_body, 0)
    _zero_rows(zero_v, RCH, 1)
    for k in range(NRCH):
        pltpu.sync_copy(zero_v, acc_sh.at[pl.ds(s * ROWS_PER_TILE + k * RCH, RCH)])
    plsc.subcore_barrier()
    base = (c * NS + s) * E_PER_W
    def body(j, carry):
        pltpu.sync_copy(dst_hbm.at[pl.ds(base + j * CH, CH)], idx_v)
        pltpu.sync_copy(ones_v, acc_sh.at[idx_v], add=True)
        return carry
    lax.fori_loop(0, NCH, body, 0)
    plsc.subcore_barrier()
    for k in range(NRCH):
        r0 = s * ROWS_PER_TILE + k * RCH
        pltpu.sync_copy(acc_sh.at[pl.ds(r0, RCH)], out_hbm.at[c, pl.ds(r0, RCH)])


@functools.partial(
    pl.kernel,
    mesh=_mesh,
    out_type=jax.ShapeDtypeStruct((NC, N_NODES, F), jnp.float32),
    scratch_types=[
        pltpu.VMEM_SHARED((N_NODES, F), jnp.float32),
        pltpu.VMEM((CH, F), jnp.float32),
        pltpu.VMEM((RCH, F), jnp.float32),
        pltpu.VMEM((CH,), jnp.int32),
        pltpu.VMEM((CH,), jnp.int32),
        pltpu.SemaphoreType.DMA,
    ],
)
def _sc_aggregate(u_hbm, src_hbm, dst_hbm, out_hbm,
                  acc_sh, rows_v, zero_v, srcv, dstv, sem):
    c = lax.axis_index("c")
    s = lax.axis_index("s")
    _zero_rows(zero_v, RCH, F // 16)
    for k in range(NRCH):
        pltpu.sync_copy(zero_v, acc_sh.at[pl.ds(s * ROWS_PER_TILE + k * RCH, RCH)])
    plsc.subcore_barrier()
    base = (c * NS + s) * E_PER_W
    def body(j, carry):
        off = base + j * CH
        pltpu.sync_copy(src_hbm.at[pl.ds(off, CH)], srcv)
        pltpu.sync_copy(dst_hbm.at[pl.ds(off, CH)], dstv)
        pltpu.async_copy(u_hbm.at[srcv], rows_v, sem).wait()
        pltpu.sync_copy(rows_v, acc_sh.at[dstv], add=True)
        return carry
    lax.fori_loop(0, NCH, body, 0)
    plsc.subcore_barrier()
    for k in range(NRCH):
        r0 = s * ROWS_PER_TILE + k * RCH
        pltpu.sync_copy(acc_sh.at[pl.ds(r0, RCH)], out_hbm.at[c, pl.ds(r0, RCH)])


_BR = 1000


def _tc_prep_body(x_ref, degp_ref, out_ref):
    deg = degp_ref[0, :, 0:1] + degp_ref[1, :, 0:1] + 1.0
    dis = lax.rsqrt(deg)
    out_ref[...] = jnp.concatenate(
        [x_ref[...] * dis, dis, jnp.zeros((_BR, F - 129), jnp.float32)], axis=1)


def _tc_prep(x, degp):
    return pl.pallas_call(
        _tc_prep_body,
        grid=(N_NODES // _BR,),
        in_specs=[
            pl.BlockSpec((_BR, 128), lambda i: (i, 0)),
            pl.BlockSpec((NC, _BR, 16), lambda i: (0, i, 0)),
        ],
        out_specs=pl.BlockSpec((_BR, F), lambda i: (i, 0)),
        out_shape=jax.ShapeDtypeStruct((N_NODES, F), jnp.float32),
    )(x, degp)


def _tc_mid_body(p1_ref, u0_ref, degp_ref, out_ref):
    deg = degp_ref[0, :, 0:1] + degp_ref[1, :, 0:1] + 1.0
    dis = lax.rsqrt(deg)
    inv = 1.0 / deg
    w1 = p1_ref[0] + p1_ref[1] + u0_ref[...]
    out_ref[...] = jnp.concatenate(
        [inv * w1[:, :128],
         jnp.zeros((_BR, 1), jnp.float32),
         dis,
         dis * w1[:, 128:129],
         jnp.zeros((_BR, F - 131), jnp.float32)], axis=1)


def _tc_mid(p1, u0, degp):
    return pl.pallas_call(
        _tc_mid_body,
        grid=(N_NODES // _BR,),
        in_specs=[
            pl.BlockSpec((NC, _BR, F), lambda i: (0, i, 0)),
            pl.BlockSpec((_BR, F), lambda i: (i, 0)),
            pl.BlockSpec((NC, _BR, 16), lambda i: (0, i, 0)),
        ],
        out_specs=pl.BlockSpec((_BR, F), lambda i: (i, 0)),
        out_shape=jax.ShapeDtypeStruct((N_NODES, F), jnp.float32),
    )(p1, u0, degp)


def _tc_weights_body(w1t_ref, w2t_ref, wht_ref, b1_ref, b2_ref, bh_ref,
                     wct_ref, cvec_ref):
    hp = jax.lax.Precision.HIGHEST
    t1 = jnp.dot(w1t_ref[...], w2t_ref[...], precision=hp)
    wct_ref[...] = jnp.dot(t1, wht_ref[...], precision=hp)
    c1 = jnp.dot(jnp.dot(b1_ref[...], w2t_ref[...], precision=hp),
                 wht_ref[...], precision=hp)
    c0 = jnp.dot(b2_ref[...], wht_ref[...], precision=hp) + bh_ref[...]
    cvec_ref[...] = jnp.concatenate([c1, c0], axis=0)


def _tc_weights(w1t, w2t, wht, b1r, b2r, bhr):
    return pl.pallas_call(
        _tc_weights_body,
        out_shape=(jax.ShapeDtypeStruct((128, 130), jnp.float32),
                   jax.ShapeDtypeStruct((2, 130), jnp.float32)),
    )(w1t, w2t, wht, b1r, b2r, bhr)


def _softplus(x):
    return jnp.maximum(x, 0.0) + jnp.log1p(jnp.exp(-jnp.abs(x)))


def _tc_final_body(p2_ref, u1_ref, wct_ref, cvec_ref,
                   mt_ref, st_ref, mz_ref, sz_ref):
    u1 = u1_ref[...]
    w2 = p2_ref[0] + p2_ref[1] + u1
    dis = u1[:, 129:130]
    a1 = u1[:, 130:131]
    z = dis * w2[:, :128]
    pre = (jnp.dot(z, wct_ref[...], precision=jax.lax.Precision.HIGHEST)
           + a1 * cvec_ref[0:1, :] + cvec_ref[1:2, :])
    mt_ref[...] = _softplus(pre[:, 0:1])
    st_ref[...] = _softplus(pre[:, 1:2])
    mz_ref[...] = pre[:, 2:66]
    sz_ref[...] = _softplus(pre[:, 66:130])


def _tc_final(p2, u1, wct, cvec):
    return pl.pallas_call(
        _tc_final_body,
        grid=(N_NODES // _BR,),
        in_specs=[
            pl.BlockSpec((NC, _BR, F), lambda i: (0, i, 0)),
            pl.BlockSpec((_BR, F), lambda i: (i, 0)),
            pl.BlockSpec((128, 130), lambda i: (0, 0)),
            pl.BlockSpec((2, 130), lambda i: (0, 0)),
        ],
        out_specs=[
            pl.BlockSpec((_BR, 1), lambda i: (i, 0)),
            pl.BlockSpec((_BR, 1), lambda i: (i, 0)),
            pl.BlockSpec((_BR, 64), lambda i: (i, 0)),
            pl.BlockSpec((_BR, 64), lambda i: (i, 0)),
        ],
        out_shape=(jax.ShapeDtypeStruct((N_NODES, 1), jnp.float32),
                   jax.ShapeDtypeStruct((N_NODES, 1), jnp.float32),
                   jax.ShapeDtypeStruct((N_NODES, 64), jnp.float32),
                   jax.ShapeDtypeStruct((N_NODES, 64), jnp.float32)),
    )(p2, u1, wct, cvec)


def kernel(data_in, edge_index, W1, b1, W2, b2,
           Wmt, bmt, Wst, bst, Wmz, bmz, Wsz, bsz):
    src = edge_index[0]
    dst = edge_index[1]

    degp = _sc_degree(dst)
    u0 = _tc_prep(data_in, degp)
    p1 = _sc_aggregate(u0, src, dst)
    u1 = _tc_mid(p1, u0, degp)
    p2 = _sc_aggregate(u1, src, dst)

    wht = jnp.concatenate([Wmt, Wst, Wmz, Wsz], axis=0).T
    bhr = jnp.concatenate([bmt, bst, bmz, bsz])[None, :]
    wct, cvec = _tc_weights(W1.T, W2.T, wht, b1[None, :], b2[None, :], bhr)

    return _tc_final(p2, u1, wct, cvec)


# speedup vs baseline: 16.2338x; 16.2338x over previous
import functools

import jax
import jax.numpy as jnp
from jax import lax
from jax.experimental import pallas as pl
from jax.experimental.pallas import tpu as pltpu
from jax.experimental.pallas import tpu_sc as plsc

N_NODES = 10000
N_EDGES = 320000
F = 144
NC, NS = 2, 16
NW = NC * NS
E_PER_W = N_EDGES // NW
CH = 80
NCH = E_PER_W // CH
RT = 624
RT_LAST = N_NODES - 15 * RT

_mesh = plsc.VectorSubcoreMesh(core_axis_name="c", subcore_axis_name="s")


def _zero_rows(buf, nrows, ncolv):
    def body(i, carry):
        for c in range(ncolv):
            buf[i, pl.ds(c * 16, 16)] = jnp.zeros((16,), jnp.float32)
        return carry
    lax.fori_loop(0, nrows, body, 0)


def _per_tile_rows(s, copyfn):
    @pl.when(s < 15)
    def _():
        copyfn(pl.multiple_of(s * RT, 8), RT)

    @pl.when(s == 15)
    def _():
        copyfn(15 * RT, RT_LAST)


@functools.partial(
    pl.kernel,
    mesh=_mesh,
    out_type=jax.ShapeDtypeStruct((NC, N_NODES, 16), jnp.float32),
    compiler_params=pltpu.CompilerParams(use_tc_tiling_on_sc=False),
    scratch_types=[
        pltpu.VMEM_SHARED((N_NODES, 16), jnp.float32),
        pltpu.VMEM((CH, 16), jnp.float32),
        pltpu.VMEM((RT_LAST, 16), jnp.float32),
        pltpu.VMEM((CH,), jnp.int32),
    ],
)
def _sc_degree(dst_hbm, out_hbm, acc_sh, ones_v, zero_v, idx_v):
    c = lax.axis_index("c")
    s = lax.axis_index("s")
    def ones_body(i, carry):
        ones_v[i, pl.ds(0, 16)] = jnp.full((16,), 1.0, jnp.float32)
        return carry
    lax.fori_loop(0, CH, ones_body, 0)
    _zero_rows(zero_v, RT_LAST, 1)
    _per_tile_rows(s, lambda r0, n: pltpu.sync_copy(
        zero_v.at[pl.ds(0, n)], acc_sh.at[pl.ds(r0, n)]))
    plsc.subcore_barrier()
    base = (c * NS + s) * E_PER_W
    def body(j, carry):
        off = pl.multiple_of(base + j * CH, 8)
        pltpu.sync_copy(dst_hbm.at[pl.ds(off, CH)], idx_v)
        pltpu.sync_copy(ones_v, acc_sh.at[idx_v], add=True)
        return carry
    lax.fori_loop(0, NCH, body, 0)
    plsc.subcore_barrier()
    _per_tile_rows(s, lambda r0, n: pltpu.sync_copy(
        acc_sh.at[pl.ds(r0, n)], out_hbm.at[c, pl.ds(r0, n)]))


@functools.partial(
    pl.kernel,
    mesh=_mesh,
    out_type=jax.ShapeDtypeStruct((NC, N_NODES, F), jnp.float32),
    compiler_params=pltpu.CompilerParams(use_tc_tiling_on_sc=False),
    scratch_types=[
        pltpu.VMEM_SHARED((N_NODES, F), jnp.float32),
        pltpu.VMEM((CH, F), jnp.float32),
        pltpu.VMEM((CH,), jnp.int32),
        pltpu.VMEM((CH,), jnp.int32),
        pltpu.SemaphoreType.DMA,
    ],
)
def _sc_aggregate(u_hbm, src_hbm, dst_hbm, out_hbm,
                  acc_sh, rows_v, srcv, dstv, sem):
    c = lax.axis_index("c")
    s = lax.axis_index("s")
    _zero_rows(rows_v, CH, F // 16)

    def zcopy(r0, n):
        for k in range(n // CH):
            pltpu.sync_copy(rows_v, acc_sh.at[pl.ds(r0 + k * CH, CH)])
        rem = n % CH
        if rem:
            pltpu.sync_copy(rows_v.at[pl.ds(0, rem)],
                            acc_sh.at[pl.ds(r0 + (n // CH) * CH, rem)])
    _per_tile_rows(s, zcopy)
    plsc.subcore_barrier()
    base = (c * NS + s) * E_PER_W
    def body(j, carry):
        off = pl.multiple_of(base + j * CH, 8)
        pltpu.sync_copy(src_hbm.at[pl.ds(off, CH)], srcv)
        pltpu.sync_copy(dst_hbm.at[pl.ds(off, CH)], dstv)
        pltpu.async_copy(u_hbm.at[srcv], rows_v, sem).wait()
        pltpu.sync_copy(rows_v, acc_sh.at[dstv], add=True)
        return carry
    lax.fori_loop(0, NCH, body, 0)
    plsc.subcore_barrier()
    _per_tile_rows(s, lambda r0, n: pltpu.sync_copy(
        acc_sh.at[pl.ds(r0, n)], out_hbm.at[c, pl.ds(r0, n)]))


_BR = 1000


def _tc_prep_body(x_ref, degp_ref, out_ref):
    deg = degp_ref[0, :, 0:1] + degp_ref[1, :, 0:1] + 1.0
    dis = lax.rsqrt(deg)
    out_ref[...] = jnp.concatenate(
        [x_ref[...] * dis, dis, jnp.zeros((_BR, F - 129), jnp.float32)], axis=1)


def _tc_prep(x, degp):
    return pl.pallas_call(
        _tc_prep_body,
        grid=(N_NODES // _BR,),
        in_specs=[
            pl.BlockSpec((_BR, 128), lambda i: (i, 0)),
            pl.BlockSpec((NC, _BR, 16), lambda i: (0, i, 0)),
        ],
        out_specs=pl.BlockSpec((_BR, F), lambda i: (i, 0)),
        out_shape=jax.ShapeDtypeStruct((N_NODES, F), jnp.float32),
    )(x, degp)


def _tc_mid_body(p1_ref, u0_ref, degp_ref, out_ref):
    deg = degp_ref[0, :, 0:1] + degp_ref[1, :, 0:1] + 1.0
    dis = lax.rsqrt(deg)
    inv = 1.0 / deg
    w1 = p1_ref[0] + p1_ref[1] + u0_ref[...]
    out_ref[...] = jnp.concatenate(
        [inv * w1[:, :128],
         jnp.zeros((_BR, 1), jnp.float32),
         dis,
         dis * w1[:, 128:129],
         jnp.zeros((_BR, F - 131), jnp.float32)], axis=1)


def _tc_mid(p1, u0, degp):
    return pl.pallas_call(
        _tc_mid_body,
        grid=(N_NODES // _BR,),
        in_specs=[
            pl.BlockSpec((NC, _BR, F), lambda i: (0, i, 0)),
            pl.BlockSpec((_BR, F), lambda i: (i, 0)),
            pl.BlockSpec((NC, _BR, 16), lambda i: (0, i, 0)),
        ],
        out_specs=pl.BlockSpec((_BR, F), lambda i: (i, 0)),
        out_shape=jax.ShapeDtypeStruct((N_NODES, F), jnp.float32),
    )(p1, u0, degp)


def _tc_weights_body(w1t_ref, w2t_ref, wht_ref, b1_ref, b2_ref, bh_ref,
                     wct_ref, cvec_ref):
    hp = jax.lax.Precision.HIGHEST
    t1 = jnp.dot(w1t_ref[...], w2t_ref[...], precision=hp)
    wct_ref[...] = jnp.dot(t1, wht_ref[...], precision=hp)
    c1 = jnp.dot(jnp.dot(b1_ref[...], w2t_ref[...], precision=hp),
                 wht_ref[...], precision=hp)
    c0 = jnp.dot(b2_ref[...], wht_ref[...], precision=hp) + bh_ref[...]
    cvec_ref[...] = jnp.concatenate([c1, c0], axis=0)


def _tc_weights(w1t, w2t, wht, b1r, b2r, bhr):
    return pl.pallas_call(
        _tc_weights_body,
        out_shape=(jax.ShapeDtypeStruct((128, 130), jnp.float32),
                   jax.ShapeDtypeStruct((2, 130), jnp.float32)),
    )(w1t, w2t, wht, b1r, b2r, bhr)


def _softplus(x):
    return jnp.maximum(x, 0.0) + jnp.log1p(jnp.exp(-jnp.abs(x)))


def _tc_final_body(p2_ref, u1_ref, wct_ref, cvec_ref,
                   mt_ref, st_ref, mz_ref, sz_ref):
    u1 = u1_ref[...]
    w2 = p2_ref[0] + p2_ref[1] + u1
    dis = u1[:, 129:130]
    a1 = u1[:, 130:131]
    z = dis * w2[:, :128]
    pre = (jnp.dot(z, wct_ref[...], precision=jax.lax.Precision.HIGHEST)
           + a1 * cvec_ref[0:1, :] + cvec_ref[1:2, :])
    mt_ref[...] = _softplus(pre[:, 0:1])
    st_ref[...] = _softplus(pre[:, 1:2])
    mz_ref[...] = pre[:, 2:66]
    sz_ref[...] = _softplus(pre[:, 66:130])


def _tc_final(p2, u1, wct, cvec):
    return pl.pallas_call(
        _tc_final_body,
        grid=(N_NODES // _BR,),
        in_specs=[
            pl.BlockSpec((NC, _BR, F), lambda i: (0, i, 0)),
            pl.BlockSpec((_BR, F), lambda i: (i, 0)),
            pl.BlockSpec((128, 130), lambda i: (0, 0)),
            pl.BlockSpec((2, 130), lambda i: (0, 0)),
        ],
        out_specs=[
            pl.BlockSpec((_BR, 1), lambda i: (i, 0)),
            pl.BlockSpec((_BR, 1), lambda i: (i, 0)),
            pl.BlockSpec((_BR, 64), lambda i: (i, 0)),
            pl.BlockSpec((_BR, 64), lambda i: (i, 0)),
        ],
        out_shape=(jax.ShapeDtypeStruct((N_NODES, 1), jnp.float32),
                   jax.ShapeDtypeStruct((N_NODES, 1), jnp.float32),
                   jax.ShapeDtypeStruct((N_NODES, 64), jnp.float32),
                   jax.ShapeDtypeStruct((N_NODES, 64), jnp.float32)),
    )(p2, u1, wct, cvec)


def kernel(data_in, edge_index, W1, b1, W2, b2,
           Wmt, bmt, Wst, bst, Wmz, bmz, Wsz, bsz):
    src = edge_index[0]
    dst = edge_index[1]

    degp = _sc_degree(dst)
    u0 = _tc_prep(data_in, degp)
    p1 = _sc_aggregate(u0, src, dst)
    u1 = _tc_mid(p1, u0, degp)
    p2 = _sc_aggregate(u1, src, dst)

    wht = jnp.concatenate([Wmt, Wst, Wmz, Wsz], axis=0).T
    bhr = jnp.concatenate([bmt, bst, bmz, bsz])[None, :]
    wct, cvec = _tc_weights(W1.T, W2.T, wht, b1[None, :], b2[None, :], bhr)

    return _tc_final(p2, u1, wct, cvec)


# speedup vs baseline: 26.7165x; 1.6457x over previous
import functools

import jax
import jax.numpy as jnp
from jax import lax
from jax.experimental import pallas as pl
from jax.experimental.pallas import tpu as pltpu
from jax.experimental.pallas import tpu_sc as plsc

N_NODES = 10000
N_EDGES = 320000
F = 144
NC, NS = 2, 16
NW = NC * NS
E_PER_W = N_EDGES // NW
CH = 80
NCH = E_PER_W // CH
RT = 624
RT_LAST = N_NODES - 15 * RT

_mesh = plsc.VectorSubcoreMesh(core_axis_name="c", subcore_axis_name="s")


def _zero_rows(buf, nrows, ncolv):
    def body(i, carry):
        for c in range(ncolv):
            buf[i, pl.ds(c * 16, 16)] = jnp.zeros((16,), jnp.float32)
        return carry
    lax.fori_loop(0, nrows, body, 0)


def _per_tile_rows(s, copyfn):
    @pl.when(s < 15)
    def _():
        copyfn(pl.multiple_of(s * RT, 8), RT)

    @pl.when(s == 15)
    def _():
        copyfn(15 * RT, RT_LAST)


@functools.partial(
    pl.kernel,
    mesh=_mesh,
    out_type=jax.ShapeDtypeStruct((NC, N_NODES, 16), jnp.float32),
    compiler_params=pltpu.CompilerParams(use_tc_tiling_on_sc=False),
    scratch_types=[
        pltpu.VMEM_SHARED((N_NODES, 16), jnp.float32),
        pltpu.VMEM((CH, 16), jnp.float32),
        pltpu.VMEM((RT_LAST, 16), jnp.float32),
        pltpu.VMEM((NCH, CH), jnp.int32),
    ],
)
def _sc_degree(dst3_hbm, out_hbm, acc_sh, ones_v, zero_v, dstall):
    c = lax.axis_index("c")
    s = lax.axis_index("s")
    wid = c * NS + s
    pltpu.sync_copy(dst3_hbm.at[wid], dstall)
    def ones_body(i, carry):
        ones_v[i, pl.ds(0, 16)] = jnp.full((16,), 1.0, jnp.float32)
        return carry
    lax.fori_loop(0, CH, ones_body, 0)
    _zero_rows(zero_v, RT_LAST, 1)
    _per_tile_rows(s, lambda r0, n: pltpu.sync_copy(
        zero_v.at[pl.ds(0, n)], acc_sh.at[pl.ds(r0, n)]))
    plsc.subcore_barrier()
    def body(j, carry):
        pltpu.sync_copy(ones_v, acc_sh.at[dstall.at[j]], add=True)
        return carry
    lax.fori_loop(0, NCH, body, 0)
    plsc.subcore_barrier()
    _per_tile_rows(s, lambda r0, n: pltpu.sync_copy(
        acc_sh.at[pl.ds(r0, n)], out_hbm.at[c, pl.ds(r0, n)]))


@functools.partial(
    pl.kernel,
    mesh=_mesh,
    out_type=jax.ShapeDtypeStruct((NC, N_NODES, F), jnp.float32),
    compiler_params=pltpu.CompilerParams(use_tc_tiling_on_sc=False),
    scratch_types=[
        pltpu.VMEM_SHARED((N_NODES, F), jnp.float32),
        pltpu.VMEM((2, CH, F), jnp.float32),
        pltpu.VMEM((NCH, CH), jnp.int32),
        pltpu.VMEM((2, CH), jnp.int32),
        pltpu.SemaphoreType.DMA,
        pltpu.SemaphoreType.DMA,
    ],
)
def _sc_aggregate(u_hbm, src_hbm, dst3_hbm, out_hbm,
                  acc_sh, rows2, dstall, srcv2, gsem, isem):
    c = lax.axis_index("c")
    s = lax.axis_index("s")
    wid = c * NS + s
    base = wid * E_PER_W
    pltpu.sync_copy(dst3_hbm.at[wid], dstall)
    _zero_rows(rows2.at[0], CH, F // 16)

    def zcopy(r0, n):
        for k in range(n // CH):
            pltpu.sync_copy(rows2.at[0], acc_sh.at[pl.ds(r0 + k * CH, CH)])
        rem = n % CH
        if rem:
            pltpu.sync_copy(rows2.at[0, pl.ds(0, rem)],
                            acc_sh.at[pl.ds(r0 + (n // CH) * CH, rem)])
    _per_tile_rows(s, zcopy)
    plsc.subcore_barrier()

    def src_slice(j):
        return src_hbm.at[pl.ds(pl.multiple_of(base + j * CH, 8), CH)]

    pltpu.sync_copy(src_slice(0), srcv2.at[0])
    pltpu.async_copy(u_hbm.at[srcv2.at[0]], rows2.at[0], gsem)
    pltpu.async_copy(src_slice(1), srcv2.at[1], isem)

    def body(j, carry):
        p = lax.rem(j, 2)
        pn = lax.rem(j + 1, 2)

        @pl.when(j + 1 < NCH)
        def _():
            pltpu.make_async_copy(src_slice(j + 1), srcv2.at[pn], isem).wait()
            pltpu.async_copy(u_hbm.at[srcv2.at[pn]], rows2.at[pn], gsem)
        pltpu.make_async_copy(u_hbm.at[srcv2.at[p]], rows2.at[p], gsem).wait()
        pltpu.sync_copy(rows2.at[p], acc_sh.at[dstall.at[j]], add=True)

        @pl.when(j + 2 < NCH)
        def _():
            pltpu.async_copy(src_slice(j + 2), srcv2.at[p], isem)
        return carry
    lax.fori_loop(0, NCH, body, 0)
    plsc.subcore_barrier()
    _per_tile_rows(s, lambda r0, n: pltpu.sync_copy(
        acc_sh.at[pl.ds(r0, n)], out_hbm.at[c, pl.ds(r0, n)]))


_BR = 1000


def _tc_prep_body(x_ref, degp_ref, out_ref):
    deg = degp_ref[0, :, 0:1] + degp_ref[1, :, 0:1] + 1.0
    dis = lax.rsqrt(deg)
    out_ref[...] = jnp.concatenate(
        [x_ref[...] * dis, dis, jnp.zeros((_BR, F - 129), jnp.float32)], axis=1)


def _tc_prep(x, degp):
    return pl.pallas_call(
        _tc_prep_body,
        grid=(N_NODES // _BR,),
        in_specs=[
            pl.BlockSpec((_BR, 128), lambda i: (i, 0)),
            pl.BlockSpec((NC, _BR, 16), lambda i: (0, i, 0)),
        ],
        out_specs=pl.BlockSpec((_BR, F), lambda i: (i, 0)),
        out_shape=jax.ShapeDtypeStruct((N_NODES, F), jnp.float32),
    )(x, degp)


def _tc_mid_body(p1_ref, u0_ref, degp_ref, out_ref):
    deg = degp_ref[0, :, 0:1] + degp_ref[1, :, 0:1] + 1.0
    dis = lax.rsqrt(deg)
    inv = 1.0 / deg
    w1 = p1_ref[0] + p1_ref[1] + u0_ref[...]
    out_ref[...] = jnp.concatenate(
        [inv * w1[:, :128],
         jnp.zeros((_BR, 1), jnp.float32),
         dis,
         dis * w1[:, 128:129],
         jnp.zeros((_BR, F - 131), jnp.float32)], axis=1)


def _tc_mid(p1, u0, degp):
    return pl.pallas_call(
        _tc_mid_body,
        grid=(N_NODES // _BR,),
        in_specs=[
            pl.BlockSpec((NC, _BR, F), lambda i: (0, i, 0)),
            pl.BlockSpec((_BR, F), lambda i: (i, 0)),
            pl.BlockSpec((NC, _BR, 16), lambda i: (0, i, 0)),
        ],
        out_specs=pl.BlockSpec((_BR, F), lambda i: (i, 0)),
        out_shape=jax.ShapeDtypeStruct((N_NODES, F), jnp.float32),
    )(p1, u0, degp)


def _tc_weights_body(w1t_ref, w2t_ref, wht_ref, b1_ref, b2_ref, bh_ref,
                     wct_ref, cvec_ref):
    hp = jax.lax.Precision.HIGHEST
    t1 = jnp.dot(w1t_ref[...], w2t_ref[...], precision=hp)
    wct_ref[...] = jnp.dot(t1, wht_ref[...], precision=hp)
    c1 = jnp.dot(jnp.dot(b1_ref[...], w2t_ref[...], precision=hp),
                 wht_ref[...], precision=hp)
    c0 = jnp.dot(b2_ref[...], wht_ref[...], precision=hp) + bh_ref[...]
    cvec_ref[...] = jnp.concatenate([c1, c0], axis=0)


def _tc_weights(w1t, w2t, wht, b1r, b2r, bhr):
    return pl.pallas_call(
        _tc_weights_body,
        out_shape=(jax.ShapeDtypeStruct((128, 130), jnp.float32),
                   jax.ShapeDtypeStruct((2, 130), jnp.float32)),
    )(w1t, w2t, wht, b1r, b2r, bhr)


def _softplus(x):
    return jnp.maximum(x, 0.0) + jnp.log1p(jnp.exp(-jnp.abs(x)))


def _tc_final_body(p2_ref, u1_ref, wct_ref, cvec_ref,
                   mt_ref, st_ref, mz_ref, sz_ref):
    u1 = u1_ref[...]
    w2 = p2_ref[0] + p2_ref[1] + u1
    dis = u1[:, 129:130]
    a1 = u1[:, 130:131]
    z = dis * w2[:, :128]
    pre = (jnp.dot(z, wct_ref[...], precision=jax.lax.Precision.HIGHEST)
           + a1 * cvec_ref[0:1, :] + cvec_ref[1:2, :])
    mt_ref[...] = _softplus(pre[:, 0:1])
    st_ref[...] = _softplus(pre[:, 1:2])
    mz_ref[...] = pre[:, 2:66]
    sz_ref[...] = _softplus(pre[:, 66:130])


def _tc_final(p2, u1, wct, cvec):
    return pl.pallas_call(
        _tc_final_body,
        grid=(N_NODES // _BR,),
        in_specs=[
            pl.BlockSpec((NC, _BR, F), lambda i: (0, i, 0)),
            pl.BlockSpec((_BR, F), lambda i: (i, 0)),
            pl.BlockSpec((128, 130), lambda i: (0, 0)),
            pl.BlockSpec((2, 130), lambda i: (0, 0)),
        ],
        out_specs=[
            pl.BlockSpec((_BR, 1), lambda i: (i, 0)),
            pl.BlockSpec((_BR, 1), lambda i: (i, 0)),
            pl.BlockSpec((_BR, 64), lambda i: (i, 0)),
            pl.BlockSpec((_BR, 64), lambda i: (i, 0)),
        ],
        out_shape=(jax.ShapeDtypeStruct((N_NODES, 1), jnp.float32),
                   jax.ShapeDtypeStruct((N_NODES, 1), jnp.float32),
                   jax.ShapeDtypeStruct((N_NODES, 64), jnp.float32),
                   jax.ShapeDtypeStruct((N_NODES, 64), jnp.float32)),
    )(p2, u1, wct, cvec)


def kernel(data_in, edge_index, W1, b1, W2, b2,
           Wmt, bmt, Wst, bst, Wmz, bmz, Wsz, bsz):
    src = edge_index[0]
    dst3 = edge_index[1].reshape(NW, NCH, CH)

    degp = _sc_degree(dst3)
    u0 = _tc_prep(data_in, degp)
    p1 = _sc_aggregate(u0, src, dst3)
    u1 = _tc_mid(p1, u0, degp)
    p2 = _sc_aggregate(u1, src, dst3)

    wht = jnp.concatenate([Wmt, Wst, Wmz, Wsz], axis=0).T
    bhr = jnp.concatenate([bmt, bst, bmz, bsz])[None, :]
    wct, cvec = _tc_weights(W1.T, W2.T, wht, b1[None, :], b2[None, :], bhr)

    return _tc_final(p2, u1, wct, cvec)


# speedup vs baseline: 30.6547x; 1.1474x over previous
import functools

import jax
import jax.numpy as jnp
from jax import lax
from jax.experimental import pallas as pl
from jax.experimental.pallas import tpu as pltpu
from jax.experimental.pallas import tpu_sc as plsc

N_NODES = 10000
N_EDGES = 320000
F = 144
NC, NS = 2, 16
NW = NC * NS
E_PER_W = N_EDGES // NW
CH = 80
NCH = E_PER_W // CH
RT = 624
RT_LAST = N_NODES - 15 * RT

_mesh = plsc.VectorSubcoreMesh(core_axis_name="c", subcore_axis_name="s")


def _zero_rows(buf, nrows, ncolv):
    def body(i, carry):
        for c in range(ncolv):
            buf[i, pl.ds(c * 16, 16)] = jnp.zeros((16,), jnp.float32)
        return carry
    lax.fori_loop(0, nrows, body, 0)


def _per_tile_rows(s, copyfn):
    @pl.when(s < 15)
    def _():
        copyfn(pl.multiple_of(s * RT, 8), RT)

    @pl.when(s == 15)
    def _():
        copyfn(15 * RT, RT_LAST)


@functools.partial(
    pl.kernel,
    mesh=_mesh,
    out_type=jax.ShapeDtypeStruct((NC, N_NODES, 16), jnp.float32),
    compiler_params=pltpu.CompilerParams(use_tc_tiling_on_sc=False),
    scratch_types=[
        pltpu.VMEM_SHARED((N_NODES, 16), jnp.float32),
        pltpu.VMEM((CH, 16), jnp.float32),
        pltpu.VMEM((RT_LAST, 16), jnp.float32),
        pltpu.VMEM((NCH, CH), jnp.int32),
    ],
)
def _sc_degree(dst3_hbm, out_hbm, acc_sh, ones_v, zero_v, dstall):
    c = lax.axis_index("c")
    s = lax.axis_index("s")
    wid = c * NS + s
    pltpu.sync_copy(dst3_hbm.at[wid], dstall)
    def ones_body(i, carry):
        ones_v[i, pl.ds(0, 16)] = jnp.full((16,), 1.0, jnp.float32)
        return carry
    lax.fori_loop(0, CH, ones_body, 0)
    _zero_rows(zero_v, RT_LAST, 1)
    _per_tile_rows(s, lambda r0, n: pltpu.sync_copy(
        zero_v.at[pl.ds(0, n)], acc_sh.at[pl.ds(r0, n)]))
    plsc.subcore_barrier()
    def body(j, carry):
        pltpu.sync_copy(ones_v, acc_sh.at[dstall.at[j]], add=True)
        return carry
    lax.fori_loop(0, NCH, body, 0)
    plsc.subcore_barrier()
    _per_tile_rows(s, lambda r0, n: pltpu.sync_copy(
        acc_sh.at[pl.ds(r0, n)], out_hbm.at[c, pl.ds(r0, n)]))


@functools.partial(
    pl.kernel,
    mesh=_mesh,
    out_type=jax.ShapeDtypeStruct((NC, N_NODES, F), jnp.float32),
    compiler_params=pltpu.CompilerParams(use_tc_tiling_on_sc=False),
    scratch_types=[
        pltpu.VMEM_SHARED((N_NODES, F), jnp.float32),
        pltpu.VMEM((2, CH, F), jnp.float32),
        pltpu.VMEM((NCH, CH), jnp.int32),
        pltpu.VMEM((2, CH), jnp.int32),
        pltpu.SemaphoreType.DMA,
        pltpu.SemaphoreType.DMA,
        pltpu.SemaphoreType.DMA,
    ],
)
def _sc_aggregate(u_hbm, src_hbm, dst3_hbm, out_hbm,
                  acc_sh, rows2, dstall, srcv2, gsem, isem, ssem):
    c = lax.axis_index("c")
    s = lax.axis_index("s")
    wid = c * NS + s
    base = wid * E_PER_W
    pltpu.sync_copy(dst3_hbm.at[wid], dstall)
    _zero_rows(rows2.at[0], CH, F // 16)

    def zcopy(r0, n):
        for k in range(n // CH):
            pltpu.sync_copy(rows2.at[0], acc_sh.at[pl.ds(r0 + k * CH, CH)])
        rem = n % CH
        if rem:
            pltpu.sync_copy(rows2.at[0, pl.ds(0, rem)],
                            acc_sh.at[pl.ds(r0 + (n // CH) * CH, rem)])
    _per_tile_rows(s, zcopy)
    plsc.subcore_barrier()

    def src_slice(j):
        return src_hbm.at[pl.ds(pl.multiple_of(base + j * CH, 8), CH)]

    pltpu.sync_copy(src_slice(0), srcv2.at[0])
    pltpu.async_copy(u_hbm.at[srcv2.at[0]], rows2.at[0], gsem)
    pltpu.async_copy(src_slice(1), srcv2.at[1], isem)

    def scat_wait():
        pltpu.make_async_copy(rows2.at[0], acc_sh.at[dstall.at[0]], ssem).wait()

    def body(j, carry):
        p = lax.rem(j, 2)
        pn = lax.rem(j + 1, 2)

        @pl.when(j + 1 < NCH)
        def _():
            pltpu.make_async_copy(src_slice(j + 1), srcv2.at[pn], isem).wait()

            @pl.when(j >= 1)
            def _():
                scat_wait()
            pltpu.async_copy(u_hbm.at[srcv2.at[pn]], rows2.at[pn], gsem)
        pltpu.make_async_copy(u_hbm.at[srcv2.at[p]], rows2.at[p], gsem).wait()
        pltpu.async_copy(rows2.at[p], acc_sh.at[dstall.at[j]], ssem, add=True)

        @pl.when(j + 2 < NCH)
        def _():
            pltpu.async_copy(src_slice(j + 2), srcv2.at[p], isem)
        return carry
    lax.fori_loop(0, NCH, body, 0)
    scat_wait()
    scat_wait()
    plsc.subcore_barrier()
    _per_tile_rows(s, lambda r0, n: pltpu.sync_copy(
        acc_sh.at[pl.ds(r0, n)], out_hbm.at[c, pl.ds(r0, n)]))


_BR = 1000


def _tc_prep_body(x_ref, degp_ref, out_ref):
    deg = degp_ref[0, :, 0:1] + degp_ref[1, :, 0:1] + 1.0
    dis = lax.rsqrt(deg)
    out_ref[...] = jnp.concatenate(
        [x_ref[...] * dis, dis, jnp.zeros((_BR, F - 129), jnp.float32)], axis=1)


def _tc_prep(x, degp):
    return pl.pallas_call(
        _tc_prep_body,
        grid=(N_NODES // _BR,),
        in_specs=[
            pl.BlockSpec((_BR, 128), lambda i: (i, 0)),
            pl.BlockSpec((NC, _BR, 16), lambda i: (0, i, 0)),
        ],
        out_specs=pl.BlockSpec((_BR, F), lambda i: (i, 0)),
        out_shape=jax.ShapeDtypeStruct((N_NODES, F), jnp.float32),
    )(x, degp)


def _tc_mid_body(p1_ref, u0_ref, degp_ref, out_ref):
    deg = degp_ref[0, :, 0:1] + degp_ref[1, :, 0:1] + 1.0
    dis = lax.rsqrt(deg)
    inv = 1.0 / deg
    w1 = p1_ref[0] + p1_ref[1] + u0_ref[...]
    out_ref[...] = jnp.concatenate(
        [inv * w1[:, :128],
         jnp.zeros((_BR, 1), jnp.float32),
         dis,
         dis * w1[:, 128:129],
         jnp.zeros((_BR, F - 131), jnp.float32)], axis=1)


def _tc_mid(p1, u0, degp):
    return pl.pallas_call(
        _tc_mid_body,
        grid=(N_NODES // _BR,),
        in_specs=[
            pl.BlockSpec((NC, _BR, F), lambda i: (0, i, 0)),
            pl.BlockSpec((_BR, F), lambda i: (i, 0)),
            pl.BlockSpec((NC, _BR, 16), lambda i: (0, i, 0)),
        ],
        out_specs=pl.BlockSpec((_BR, F), lambda i: (i, 0)),
        out_shape=jax.ShapeDtypeStruct((N_NODES, F), jnp.float32),
    )(p1, u0, degp)


def _softplus(x):
    return jnp.maximum(x, 0.0) + jnp.log1p(jnp.exp(-jnp.abs(x)))


def _tc_final_body(p2_ref, u1_ref, w1t_ref, w2t_ref, wht_ref,
                   b1_ref, b2_ref, bh_ref,
                   mt_ref, st_ref, mz_ref, sz_ref):
    hp = jax.lax.Precision.HIGHEST
    t1 = jnp.dot(w1t_ref[...], w2t_ref[...], precision=hp)
    wct = jnp.dot(t1, wht_ref[...], precision=hp)
    c1 = jnp.dot(jnp.dot(b1_ref[...], w2t_ref[...], precision=hp),
                 wht_ref[...], precision=hp)
    c0 = jnp.dot(b2_ref[...], wht_ref[...], precision=hp) + bh_ref[...]
    u1 = u1_ref[...]
    w2 = p2_ref[0] + p2_ref[1] + u1
    dis = u1[:, 129:130]
    a1 = u1[:, 130:131]
    z = dis * w2[:, :128]
    pre = jnp.dot(z, wct, precision=hp) + a1 * c1 + c0
    mt_ref[...] = _softplus(pre[:, 0:1])
    st_ref[...] = _softplus(pre[:, 1:2])
    mz_ref[...] = pre[:, 2:66]
    sz_ref[...] = _softplus(pre[:, 66:130])


def _tc_final(p2, u1, w1t, w2t, wht, b1r, b2r, bhr):
    return pl.pallas_call(
        _tc_final_body,
        grid=(N_NODES // _BR,),
        in_specs=[
            pl.BlockSpec((NC, _BR, F), lambda i: (0, i, 0)),
            pl.BlockSpec((_BR, F), lambda i: (i, 0)),
            pl.BlockSpec((128, 500), lambda i: (0, 0)),
            pl.BlockSpec((500, 250), lambda i: (0, 0)),
            pl.BlockSpec((250, 130), lambda i: (0, 0)),
            pl.BlockSpec((1, 500), lambda i: (0, 0)),
            pl.BlockSpec((1, 250), lambda i: (0, 0)),
            pl.BlockSpec((1, 130), lambda i: (0, 0)),
        ],
        out_specs=[
            pl.BlockSpec((_BR, 1), lambda i: (i, 0)),
            pl.BlockSpec((_BR, 1), lambda i: (i, 0)),
            pl.BlockSpec((_BR, 64), lambda i: (i, 0)),
            pl.BlockSpec((_BR, 64), lambda i: (i, 0)),
        ],
        out_shape=(jax.ShapeDtypeStruct((N_NODES, 1), jnp.float32),
                   jax.ShapeDtypeStruct((N_NODES, 1), jnp.float32),
                   jax.ShapeDtypeStruct((N_NODES, 64), jnp.float32),
                   jax.ShapeDtypeStruct((N_NODES, 64), jnp.float32)),
    )(p2, u1, w1t, w2t, wht, b1r, b2r, bhr)


def kernel(data_in, edge_index, W1, b1, W2, b2,
           Wmt, bmt, Wst, bst, Wmz, bmz, Wsz, bsz):
    src = edge_index[0]
    dst3 = edge_index[1].reshape(NW, NCH, CH)

    degp = _sc_degree(dst3)
    u0 = _tc_prep(data_in, degp)
    p1 = _sc_aggregate(u0, src, dst3)
    u1 = _tc_mid(p1, u0, degp)
    p2 = _sc_aggregate(u1, src, dst3)

    wht = jnp.concatenate([Wmt, Wst, Wmz, Wsz], axis=0).T
    bhr = jnp.concatenate([bmt, bst, bmz, bsz])[None, :]
    return _tc_final(p2, u1, W1.T, W2.T, wht, b1[None, :], b2[None, :], bhr)
